# baseline (speedup 1.0000x reference)
import dataclasses
import functools

import jax
import jax.numpy as jnp
from jax import lax
from jax.experimental import pallas as pl
from jax.experimental.pallas import tpu as pltpu
from jax.experimental.pallas import tpu_sc as plsc

NBINS = 256
NWORKERS = 32


def _minmax_body(x_ref, o_ref, acc_ref):
    i = pl.program_id(0)
    bm = jnp.min(x_ref[...])
    bx = jnp.max(x_ref[...])

    @pl.when(i == 0)
    def _init():
        acc_ref[0, 0] = bm
        acc_ref[0, 1] = bx

    @pl.when(i > 0)
    def _acc():
        acc_ref[0, 0] = jnp.minimum(acc_ref[0, 0], bm)
        acc_ref[0, 1] = jnp.maximum(acc_ref[0, 1], bx)

    mn = acc_ref[0, 0]
    mx = acc_ref[0, 1]
    width = (mx - mn) / jnp.float32(NBINS)
    row = lax.broadcasted_iota(jnp.int32, (8, 128), 0)
    o_ref[...] = jnp.where(row == 0, mn, width)


def _minmax(x_flat, nrows, npix):
    ncols = 16384
    grid = npix // ncols
    return pl.pallas_call(
        _minmax_body,
        grid=(grid,),
        in_specs=[pl.BlockSpec((nrows, ncols), lambda i: (0, i))],
        out_specs=pl.BlockSpec((8, 128), lambda i: (0, 0)),
        out_shape=jax.ShapeDtypeStruct((8, 128), jnp.float32),
        scratch_shapes=[pltpu.SMEM((1, 2), jnp.float32)],
    )(x_flat)


def _sc_hist_body(nrows, chunk, x_hbm, mm_hbm, hist_hbm,
                  mm_v, buf0_v, buf1_v, hist_v, sem_a, sem_b):
    wid = lax.axis_index("s") * 2 + lax.axis_index("c")
    base = wid * chunk

    zeros16 = jnp.zeros((16,), jnp.float32)

    @pl.loop(0, nrows * NBINS, step=16)
    def _zero(i):
        hist_v[pl.ds(i, 16)] = zeros16

    pltpu.sync_copy(mm_hbm, mm_v)
    mn_v = mm_v[0, pl.ds(0, 16)]
    w_v = mm_v[1, pl.ds(0, 16)]

    ones16 = jnp.ones((16,), jnp.float32)
    top = jnp.full((16,), NBINS - 1, jnp.int32)
    bot = jnp.zeros((16,), jnp.int32)

    sems = (sem_a, sem_b)
    bufs = (buf0_v, buf1_v)
    cp = pltpu.async_copy(x_hbm.at[0, pl.ds(base, chunk)], bufs[0], sems[0])
    for r in range(nrows):
        cur = r % 2
        cp.wait()
        if r + 1 < nrows:
            cp = pltpu.async_copy(
                x_hbm.at[r + 1, pl.ds(base, chunk)],
                bufs[1 - cur], sems[1 - cur])
        rbase = jnp.full((16,), r * NBINS, jnp.int32)
        bufr = bufs[cur]

        @pl.loop(0, chunk, step=128)
        def _bin(i):
            for u in range(8):
                v = bufr[pl.ds(i + u * 16, 16)]
                q = (v - mn_v) / w_v
                b = q.astype(jnp.int32)
                b = jnp.minimum(jnp.maximum(b, bot), top) + rbase
                plsc.addupdate_scatter(hist_v, [b], ones16)

    pltpu.sync_copy(hist_v, hist_hbm.at[wid])


def _sc_hist(x_flat, mm, nrows, npix):
    chunk = npix // NWORKERS
    mesh = plsc.VectorSubcoreMesh(core_axis_name="c", subcore_axis_name="s")
    cp = pltpu.CompilerParams()
    if "needs_layout_passes" in pltpu.CompilerParams.__dataclass_fields__:
        cp = dataclasses.replace(cp, needs_layout_passes=False)
    kern = functools.partial(
        pl.kernel,
        compiler_params=cp,
        out_type=jax.ShapeDtypeStruct((NWORKERS, nrows * NBINS), jnp.float32),
        mesh=mesh,
        scratch_types=[
            pltpu.VMEM((8, 128), jnp.float32),
            pltpu.VMEM((chunk,), jnp.float32),
            pltpu.VMEM((chunk,), jnp.float32),
            pltpu.VMEM((nrows * NBINS,), jnp.float32),
            pltpu.SemaphoreType.DMA,
            pltpu.SemaphoreType.DMA,
        ],
    )(functools.partial(_sc_hist_body, nrows, chunk))
    return kern(x_flat, mm)


def _make_finalize_body(nrows, npix):
    inv_n = 1.0 / float(npix)

    def body(x_ref, hist_ref, mm_ref, o_ref, thr_ref):
        i = pl.program_id(0)

        @pl.when(i == 0)
        def _thresholds():
            h = hist_ref[0]
            for k in range(1, NWORKERS):
                h = h + hist_ref[k]
            hn = h * jnp.float32(inv_n)
            tv = lax.broadcasted_iota(jnp.int32, (1, NBINS), 1).astype(jnp.float32)
            bi = lax.broadcasted_iota(jnp.int32, (NBINS, NBINS), 0)
            ti = lax.broadcasted_iota(jnp.int32, (NBINS, NBINS), 1)
            cum = (bi <= ti).astype(jnp.float32)
            w_bg = jnp.dot(hn, cum, precision=lax.Precision.HIGHEST)
            s_bg = jnp.dot(hn * tv, cum, precision=lax.Precision.HIGHEST)
            total = s_bg[:, NBINS - 1:NBINS]
            w_fg = 1.0 - w_bg
            valid = (w_bg != 0.0) & (w_fg != 0.0)
            safe_w_bg = jnp.where(valid, w_bg, jnp.float32(1.0))
            safe_w_fg = jnp.where(valid, w_fg, jnp.float32(1.0))
            mean_bg = s_bg / safe_w_bg
            mean_fg = (total - s_bg) / safe_w_fg
            icv = w_bg * w_fg * (mean_bg - mean_fg) ** 2
            icv = jnp.where(valid, icv, -jnp.inf)
            mxv = jnp.max(icv, axis=1, keepdims=True)
            ii = lax.broadcasted_iota(jnp.int32, (nrows, NBINS), 1)
            t_best = jnp.min(jnp.where(icv == mxv, ii, NBINS),
                             axis=1, keepdims=True)
            any_valid = jnp.max(valid.astype(jnp.int32), axis=1,
                                keepdims=True) > 0
            mn = mm_ref[0:1, 0:1]
            width = mm_ref[1:2, 0:1]
            thr = mn + (t_best + 1).astype(jnp.float32) * width
            thr = jnp.where(any_valid, thr, jnp.float32(0.0))
            thr_ref[...] = jnp.broadcast_to(thr, (nrows, 128))

        xb = x_ref[...]
        thr = thr_ref[:, 0:1]
        o_ref[...] = jnp.where(xb <= thr, jnp.float32(0.0), xb)

    return body


def _finalize(x_flat, hist_parts, mm, nrows, npix):
    ncols = 8192
    grid = npix // ncols
    return pl.pallas_call(
        _make_finalize_body(nrows, npix),
        grid=(grid,),
        in_specs=[
            pl.BlockSpec((nrows, ncols), lambda i: (0, i)),
            pl.BlockSpec((NWORKERS, nrows, NBINS), lambda i: (0, 0, 0)),
            pl.BlockSpec((8, 128), lambda i: (0, 0)),
        ],
        out_specs=pl.BlockSpec((nrows, ncols), lambda i: (0, i)),
        out_shape=jax.ShapeDtypeStruct((nrows, npix), jnp.float32),
        scratch_shapes=[pltpu.VMEM((nrows, 128), jnp.float32)],
    )(x_flat, hist_parts, mm)


def kernel(x):
    b, c, h, w = x.shape
    nrows = b * c
    npix = h * w
    x_flat = x.reshape(nrows, npix)
    mm = _minmax(x_flat, nrows, npix)
    hist_parts = _sc_hist(x_flat, mm, nrows, npix)
    out = _finalize(x_flat, hist_parts.reshape(NWORKERS, nrows, NBINS),
                    mm, nrows, npix)
    return out.reshape(x.shape)


# speedup vs baseline: 1.0009x; 1.0009x over previous
import dataclasses
import functools

import jax
import jax.numpy as jnp
from jax import lax
from jax.experimental import pallas as pl
from jax.experimental.pallas import tpu as pltpu
from jax.experimental.pallas import tpu_sc as plsc

NBINS = 256
NWORKERS = 32


def _minmax_body(x_ref, o_ref, acc_ref):
    i = pl.program_id(0)
    bm = jnp.min(x_ref[...])
    bx = jnp.max(x_ref[...])

    @pl.when(i == 0)
    def _init():
        acc_ref[0, 0] = bm
        acc_ref[0, 1] = bx

    @pl.when(i > 0)
    def _acc():
        acc_ref[0, 0] = jnp.minimum(acc_ref[0, 0], bm)
        acc_ref[0, 1] = jnp.maximum(acc_ref[0, 1], bx)

    mn = acc_ref[0, 0]
    mx = acc_ref[0, 1]
    width = (mx - mn) / jnp.float32(NBINS)
    row = lax.broadcasted_iota(jnp.int32, (8, 128), 0)
    o_ref[...] = jnp.where(row == 0, mn, width)


def _minmax(x_flat, nrows, npix):
    ncols = 16384
    grid = npix // ncols
    return pl.pallas_call(
        _minmax_body,
        grid=(grid,),
        in_specs=[pl.BlockSpec((nrows, ncols), lambda i: (0, i))],
        out_specs=pl.BlockSpec((8, 128), lambda i: (0, 0)),
        out_shape=jax.ShapeDtypeStruct((8, 128), jnp.float32),
        scratch_shapes=[pltpu.SMEM((1, 2), jnp.float32)],
    )(x_flat)


def _sc_hist_body(nrows, chunk, x_hbm, mm_hbm, hist_hbm,
                  mm_v, buf0_v, buf1_v, hist_v, sem_a, sem_b):
    wid = lax.axis_index("s") * 2 + lax.axis_index("c")
    base = wid * chunk

    zeros16 = jnp.zeros((16,), jnp.float32)

    @pl.loop(0, nrows * NBINS, step=16)
    def _zero(i):
        hist_v[pl.ds(i, 16)] = zeros16

    pltpu.sync_copy(mm_hbm, mm_v)
    mn_v = mm_v[0, pl.ds(0, 16)]
    w_v = mm_v[1, pl.ds(0, 16)]
    rw_v = jnp.float32(1.0) / w_v

    ones16 = jnp.ones((16,), jnp.float32)
    top = jnp.full((16,), NBINS - 1, jnp.int32)
    bot = jnp.zeros((16,), jnp.int32)

    sems = (sem_a, sem_b)
    bufs = (buf0_v, buf1_v)
    cp = pltpu.async_copy(x_hbm.at[0, pl.ds(base, chunk)], bufs[0], sems[0])
    for r in range(nrows):
        cur = r % 2
        cp.wait()
        if r + 1 < nrows:
            cp = pltpu.async_copy(
                x_hbm.at[r + 1, pl.ds(base, chunk)],
                bufs[1 - cur], sems[1 - cur])
        rbase = jnp.full((16,), r * NBINS, jnp.int32)
        bufr = bufs[cur]

        @pl.loop(0, chunk, step=128)
        def _bin(i):
            for u in range(8):
                v = bufr[pl.ds(i + u * 16, 16)]
                q = (v - mn_v) * rw_v
                b = q.astype(jnp.int32)
                b = jnp.minimum(jnp.maximum(b, bot), top) + rbase
                plsc.addupdate_scatter(hist_v, [b], ones16)

    pltpu.sync_copy(hist_v, hist_hbm.at[wid])


def _sc_hist(x_flat, mm, nrows, npix):
    chunk = npix // NWORKERS
    mesh = plsc.VectorSubcoreMesh(core_axis_name="c", subcore_axis_name="s")
    cp = pltpu.CompilerParams()
    if "needs_layout_passes" in pltpu.CompilerParams.__dataclass_fields__:
        cp = dataclasses.replace(cp, needs_layout_passes=False)
    kern = functools.partial(
        pl.kernel,
        compiler_params=cp,
        out_type=jax.ShapeDtypeStruct((NWORKERS, nrows * NBINS), jnp.float32),
        mesh=mesh,
        scratch_types=[
            pltpu.VMEM((8, 128), jnp.float32),
            pltpu.VMEM((chunk,), jnp.float32),
            pltpu.VMEM((chunk,), jnp.float32),
            pltpu.VMEM((nrows * NBINS,), jnp.float32),
            pltpu.SemaphoreType.DMA,
            pltpu.SemaphoreType.DMA,
        ],
    )(functools.partial(_sc_hist_body, nrows, chunk))
    return kern(x_flat, mm)


def _make_finalize_body(nrows, npix):
    inv_n = 1.0 / float(npix)

    def body(x_ref, hist_ref, mm_ref, o_ref, thr_ref):
        i = pl.program_id(0)

        @pl.when(i == 0)
        def _thresholds():
            h = hist_ref[0]
            for k in range(1, NWORKERS):
                h = h + hist_ref[k]
            hn = h * jnp.float32(inv_n)
            tv = lax.broadcasted_iota(jnp.int32, (1, NBINS), 1).astype(jnp.float32)
            bi = lax.broadcasted_iota(jnp.int32, (NBINS, NBINS), 0)
            ti = lax.broadcasted_iota(jnp.int32, (NBINS, NBINS), 1)
            cum = (bi <= ti).astype(jnp.float32)
            w_bg = jnp.dot(hn, cum, precision=lax.Precision.HIGHEST)
            s_bg = jnp.dot(hn * tv, cum, precision=lax.Precision.HIGHEST)
            total = s_bg[:, NBINS - 1:NBINS]
            w_fg = 1.0 - w_bg
            valid = (w_bg != 0.0) & (w_fg != 0.0)
            safe_w_bg = jnp.where(valid, w_bg, jnp.float32(1.0))
            safe_w_fg = jnp.where(valid, w_fg, jnp.float32(1.0))
            mean_bg = s_bg / safe_w_bg
            mean_fg = (total - s_bg) / safe_w_fg
            icv = w_bg * w_fg * (mean_bg - mean_fg) ** 2
            icv = jnp.where(valid, icv, -jnp.inf)
            mxv = jnp.max(icv, axis=1, keepdims=True)
            ii = lax.broadcasted_iota(jnp.int32, (nrows, NBINS), 1)
            t_best = jnp.min(jnp.where(icv == mxv, ii, NBINS),
                             axis=1, keepdims=True)
            any_valid = jnp.max(valid.astype(jnp.int32), axis=1,
                                keepdims=True) > 0
            mn = mm_ref[0:1, 0:1]
            width = mm_ref[1:2, 0:1]
            thr = mn + (t_best + 1).astype(jnp.float32) * width
            thr = jnp.where(any_valid, thr, jnp.float32(0.0))
            thr_ref[...] = jnp.broadcast_to(thr, (nrows, 128))

        xb = x_ref[...]
        thr = thr_ref[:, 0:1]
        o_ref[...] = jnp.where(xb <= thr, jnp.float32(0.0), xb)

    return body


def _finalize(x_flat, hist_parts, mm, nrows, npix):
    ncols = 8192
    grid = npix // ncols
    return pl.pallas_call(
        _make_finalize_body(nrows, npix),
        grid=(grid,),
        in_specs=[
            pl.BlockSpec((nrows, ncols), lambda i: (0, i)),
            pl.BlockSpec((NWORKERS, nrows, NBINS), lambda i: (0, 0, 0)),
            pl.BlockSpec((8, 128), lambda i: (0, 0)),
        ],
        out_specs=pl.BlockSpec((nrows, ncols), lambda i: (0, i)),
        out_shape=jax.ShapeDtypeStruct((nrows, npix), jnp.float32),
        scratch_shapes=[pltpu.VMEM((nrows, 128), jnp.float32)],
    )(x_flat, hist_parts, mm)


def kernel(x):
    b, c, h, w = x.shape
    nrows = b * c
    npix = h * w
    x_flat = x.reshape(nrows, npix)
    mm = _minmax(x_flat, nrows, npix)
    hist_parts = _sc_hist(x_flat, mm, nrows, npix)
    out = _finalize(x_flat, hist_parts.reshape(NWORKERS, nrows, NBINS),
                    mm, nrows, npix)
    return out.reshape(x.shape)


# speedup vs baseline: 1.0050x; 1.0041x over previous
import dataclasses
import functools

import jax
import jax.numpy as jnp
from jax import lax
from jax.experimental import pallas as pl
from jax.experimental.pallas import tpu as pltpu
from jax.experimental.pallas import tpu_sc as plsc

NBINS = 256
NWORKERS = 32


def _minmax_body(x_ref, o_ref, amn_ref, amx_ref):
    i = pl.program_id(0)
    xb = x_ref[...]

    @pl.when(i == 0)
    def _init():
        amn_ref[...] = xb
        amx_ref[...] = xb

    @pl.when(i > 0)
    def _acc():
        amn_ref[...] = jnp.minimum(amn_ref[...], xb)
        amx_ref[...] = jnp.maximum(amx_ref[...], xb)

    @pl.when(i == pl.num_programs(0) - 1)
    def _emit():
        mn = jnp.min(amn_ref[...])
        mx = jnp.max(amx_ref[...])
        width = (mx - mn) / jnp.float32(NBINS)
        row = lax.broadcasted_iota(jnp.int32, (8, 128), 0)
        o_ref[...] = jnp.where(row == 0, mn, width)


def _minmax(x_flat, nrows, npix):
    ncols = 16384
    grid = npix // ncols
    return pl.pallas_call(
        _minmax_body,
        grid=(grid,),
        in_specs=[pl.BlockSpec((nrows, ncols), lambda i: (0, i))],
        out_specs=pl.BlockSpec((8, 128), lambda i: (0, 0)),
        out_shape=jax.ShapeDtypeStruct((8, 128), jnp.float32),
        scratch_shapes=[pltpu.VMEM((nrows, ncols), jnp.float32),
                        pltpu.VMEM((nrows, ncols), jnp.float32)],
    )(x_flat)


def _sc_hist_body(nrows, chunk, x_hbm, mm_hbm, hist_hbm,
                  mm_v, buf0_v, buf1_v, hist_v, sem_a, sem_b):
    wid = lax.axis_index("s") * 2 + lax.axis_index("c")
    base = wid * chunk

    zeros16 = jnp.zeros((16,), jnp.float32)

    @pl.loop(0, nrows * NBINS, step=16)
    def _zero(i):
        hist_v[pl.ds(i, 16)] = zeros16

    pltpu.sync_copy(mm_hbm, mm_v)
    mn_v = mm_v[0, pl.ds(0, 16)]
    w_v = mm_v[1, pl.ds(0, 16)]
    rw_v = jnp.float32(1.0) / w_v

    ones16 = jnp.ones((16,), jnp.float32)
    top = jnp.full((16,), NBINS - 1, jnp.int32)
    bot = jnp.zeros((16,), jnp.int32)

    sems = (sem_a, sem_b)
    bufs = (buf0_v, buf1_v)
    cp = pltpu.async_copy(x_hbm.at[0, pl.ds(base, chunk)], bufs[0], sems[0])
    for r in range(nrows):
        cur = r % 2
        cp.wait()
        if r + 1 < nrows:
            cp = pltpu.async_copy(
                x_hbm.at[r + 1, pl.ds(base, chunk)],
                bufs[1 - cur], sems[1 - cur])
        rbase = jnp.full((16,), r * NBINS, jnp.int32)
        bufr = bufs[cur]

        @pl.loop(0, chunk, step=128)
        def _bin(i):
            for u in range(8):
                v = bufr[pl.ds(i + u * 16, 16)]
                q = (v - mn_v) / w_v
                b = q.astype(jnp.int32)
                b = jnp.minimum(jnp.maximum(b, bot), top) + rbase
                plsc.addupdate_scatter(hist_v, [b], ones16)

    pltpu.sync_copy(hist_v, hist_hbm.at[wid])


def _sc_hist(x_flat, mm, nrows, npix):
    chunk = npix // NWORKERS
    mesh = plsc.VectorSubcoreMesh(core_axis_name="c", subcore_axis_name="s")
    cp = pltpu.CompilerParams()
    if "needs_layout_passes" in pltpu.CompilerParams.__dataclass_fields__:
        cp = dataclasses.replace(cp, needs_layout_passes=False)
    kern = functools.partial(
        pl.kernel,
        compiler_params=cp,
        out_type=jax.ShapeDtypeStruct((NWORKERS, nrows * NBINS), jnp.float32),
        mesh=mesh,
        scratch_types=[
            pltpu.VMEM((8, 128), jnp.float32),
            pltpu.VMEM((chunk,), jnp.float32),
            pltpu.VMEM((chunk,), jnp.float32),
            pltpu.VMEM((nrows * NBINS,), jnp.float32),
            pltpu.SemaphoreType.DMA,
            pltpu.SemaphoreType.DMA,
        ],
    )(functools.partial(_sc_hist_body, nrows, chunk))
    return kern(x_flat, mm)


def _make_finalize_body(nrows, npix):
    inv_n = 1.0 / float(npix)

    def body(x_ref, hist_ref, mm_ref, o_ref, thr_ref):
        i = pl.program_id(0)

        @pl.when(i == 0)
        def _thresholds():
            h = hist_ref[0]
            for k in range(1, NWORKERS):
                h = h + hist_ref[k]
            hn = h * jnp.float32(inv_n)
            tv = lax.broadcasted_iota(jnp.int32, (1, NBINS), 1).astype(jnp.float32)
            bi = lax.broadcasted_iota(jnp.int32, (NBINS, NBINS), 0)
            ti = lax.broadcasted_iota(jnp.int32, (NBINS, NBINS), 1)
            cum = (bi <= ti).astype(jnp.float32)
            w_bg = jnp.dot(hn, cum, precision=lax.Precision.HIGHEST)
            s_bg = jnp.dot(hn * tv, cum, precision=lax.Precision.HIGHEST)
            total = s_bg[:, NBINS - 1:NBINS]
            w_fg = 1.0 - w_bg
            valid = (w_bg != 0.0) & (w_fg != 0.0)
            safe_w_bg = jnp.where(valid, w_bg, jnp.float32(1.0))
            safe_w_fg = jnp.where(valid, w_fg, jnp.float32(1.0))
            mean_bg = s_bg / safe_w_bg
            mean_fg = (total - s_bg) / safe_w_fg
            icv = w_bg * w_fg * (mean_bg - mean_fg) ** 2
            icv = jnp.where(valid, icv, -jnp.inf)
            mxv = jnp.max(icv, axis=1, keepdims=True)
            ii = lax.broadcasted_iota(jnp.int32, (nrows, NBINS), 1)
            t_best = jnp.min(jnp.where(icv == mxv, ii, NBINS),
                             axis=1, keepdims=True)
            any_valid = jnp.max(valid.astype(jnp.int32), axis=1,
                                keepdims=True) > 0
            mn = mm_ref[0:1, 0:1]
            width = mm_ref[1:2, 0:1]
            thr = mn + (t_best + 1).astype(jnp.float32) * width
            thr = jnp.where(any_valid, thr, jnp.float32(0.0))
            thr_ref[...] = jnp.broadcast_to(thr, (nrows, 128))

        xb = x_ref[...]
        thr = thr_ref[:, 0:1]
        o_ref[...] = jnp.where(xb <= thr, jnp.float32(0.0), xb)

    return body


def _finalize(x_flat, hist_parts, mm, nrows, npix):
    ncols = 8192
    grid = npix // ncols
    return pl.pallas_call(
        _make_finalize_body(nrows, npix),
        grid=(grid,),
        in_specs=[
            pl.BlockSpec((nrows, ncols), lambda i: (0, i)),
            pl.BlockSpec((NWORKERS, nrows, NBINS), lambda i: (0, 0, 0)),
            pl.BlockSpec((8, 128), lambda i: (0, 0)),
        ],
        out_specs=pl.BlockSpec((nrows, ncols), lambda i: (0, i)),
        out_shape=jax.ShapeDtypeStruct((nrows, npix), jnp.float32),
        scratch_shapes=[pltpu.VMEM((nrows, 128), jnp.float32)],
    )(x_flat, hist_parts, mm)


def kernel(x):
    b, c, h, w = x.shape
    nrows = b * c
    npix = h * w
    x_flat = x.reshape(nrows, npix)
    mm = _minmax(x_flat, nrows, npix)
    hist_parts = _sc_hist(x_flat, mm, nrows, npix)
    out = _finalize(x_flat, hist_parts.reshape(NWORKERS, nrows, NBINS),
                    mm, nrows, npix)
    return out.reshape(x.shape)


# speedup vs baseline: 1.0307x; 1.0256x over previous
import dataclasses
import functools

import jax
import jax.numpy as jnp
from jax import lax
from jax.experimental import pallas as pl
from jax.experimental.pallas import tpu as pltpu
from jax.experimental.pallas import tpu_sc as plsc

NBINS = 256
NWORKERS = 32


def _minmax_body(x_ref, o_ref, amn_ref, amx_ref):
    i = pl.program_id(0)
    xb = x_ref[...]

    @pl.when(i == 0)
    def _init():
        amn_ref[...] = xb
        amx_ref[...] = xb

    @pl.when(i > 0)
    def _acc():
        amn_ref[...] = jnp.minimum(amn_ref[...], xb)
        amx_ref[...] = jnp.maximum(amx_ref[...], xb)

    @pl.when(i == pl.num_programs(0) - 1)
    def _emit():
        mn = jnp.min(amn_ref[...])
        mx = jnp.max(amx_ref[...])
        width = (mx - mn) / jnp.float32(NBINS)
        row = lax.broadcasted_iota(jnp.int32, (8, 128), 0)
        o_ref[...] = jnp.where(row == 0, mn, width)


def _minmax(x_flat, nrows, npix):
    ncols = 16384
    grid = npix // ncols
    return pl.pallas_call(
        _minmax_body,
        grid=(grid,),
        in_specs=[pl.BlockSpec((nrows, ncols), lambda i: (0, i))],
        out_specs=pl.BlockSpec((8, 128), lambda i: (0, 0)),
        out_shape=jax.ShapeDtypeStruct((8, 128), jnp.float32),
        scratch_shapes=[pltpu.VMEM((nrows, ncols), jnp.float32),
                        pltpu.VMEM((nrows, ncols), jnp.float32)],
    )(x_flat)


def _sc_hist_body(nrows, chunk, x_hbm, mm_hbm, hist_hbm,
                  mm_v, buf0_v, buf1_v, hist_v, sem_a, sem_b):
    wid = lax.axis_index("s") * 2 + lax.axis_index("c")
    base = wid * chunk

    zeros16 = jnp.zeros((16,), jnp.float32)

    @pl.loop(0, nrows * NBINS, step=16)
    def _zero(i):
        hist_v[pl.ds(i, 16)] = zeros16

    pltpu.sync_copy(mm_hbm, mm_v)
    mn_v = mm_v[0, pl.ds(0, 16)]
    w_v = mm_v[1, pl.ds(0, 16)]
    rw_v = jnp.float32(1.0) / w_v

    ones16 = jnp.ones((16,), jnp.float32)
    top = jnp.full((16,), NBINS - 1, jnp.int32)
    bot = jnp.zeros((16,), jnp.int32)

    sems = (sem_a, sem_b)
    bufs = (buf0_v, buf1_v)
    cp = pltpu.async_copy(x_hbm.at[0, pl.ds(base, chunk)], bufs[0], sems[0])
    for r in range(nrows):
        cur = r % 2
        cp.wait()
        if r + 1 < nrows:
            cp = pltpu.async_copy(
                x_hbm.at[r + 1, pl.ds(base, chunk)],
                bufs[1 - cur], sems[1 - cur])
        rbase = jnp.full((16,), r * NBINS, jnp.int32)
        bufr = bufs[cur]

        @pl.loop(0, chunk, step=128)
        def _bin(i):
            for u in range(8):
                v = bufr[pl.ds(i + u * 16, 16)]
                q = (v - mn_v) / w_v
                b = q.astype(jnp.int32)
                b = jnp.minimum(jnp.maximum(b, bot), top) + rbase
                plsc.addupdate_scatter(hist_v, [b], ones16)

    pltpu.sync_copy(hist_v, hist_hbm.at[wid])


def _sc_hist(x_flat, mm, nrows, npix):
    chunk = npix // NWORKERS
    mesh = plsc.VectorSubcoreMesh(core_axis_name="c", subcore_axis_name="s")
    cp = pltpu.CompilerParams()
    if "needs_layout_passes" in pltpu.CompilerParams.__dataclass_fields__:
        cp = dataclasses.replace(cp, needs_layout_passes=False)
    kern = functools.partial(
        pl.kernel,
        compiler_params=cp,
        out_type=jax.ShapeDtypeStruct((NWORKERS, nrows * NBINS), jnp.float32),
        mesh=mesh,
        scratch_types=[
            pltpu.VMEM((8, 128), jnp.float32),
            pltpu.VMEM((chunk,), jnp.float32),
            pltpu.VMEM((chunk,), jnp.float32),
            pltpu.VMEM((nrows * NBINS,), jnp.float32),
            pltpu.SemaphoreType.DMA,
            pltpu.SemaphoreType.DMA,
        ],
    )(functools.partial(_sc_hist_body, nrows, chunk))
    return kern(x_flat, mm)


def _make_thresholds_body(nrows, npix):
    inv_n = 1.0 / float(npix)

    def body(hist_ref, mm_ref, thr_ref):
        h = hist_ref[0]
        for k in range(1, NWORKERS):
            h = h + hist_ref[k]
        hn = h * jnp.float32(inv_n)
        tv = lax.broadcasted_iota(jnp.int32, (1, NBINS), 1).astype(jnp.float32)
        bi = lax.broadcasted_iota(jnp.int32, (NBINS, NBINS), 0)
        ti = lax.broadcasted_iota(jnp.int32, (NBINS, NBINS), 1)
        cum = (bi <= ti).astype(jnp.float32)
        w_bg = jnp.dot(hn, cum, precision=lax.Precision.HIGHEST)
        s_bg = jnp.dot(hn * tv, cum, precision=lax.Precision.HIGHEST)
        total = s_bg[:, NBINS - 1:NBINS]
        w_fg = 1.0 - w_bg
        valid = (w_bg != 0.0) & (w_fg != 0.0)
        safe_w_bg = jnp.where(valid, w_bg, jnp.float32(1.0))
        safe_w_fg = jnp.where(valid, w_fg, jnp.float32(1.0))
        mean_bg = s_bg / safe_w_bg
        mean_fg = (total - s_bg) / safe_w_fg
        icv = w_bg * w_fg * (mean_bg - mean_fg) ** 2
        icv = jnp.where(valid, icv, -jnp.inf)
        mxv = jnp.max(icv, axis=1, keepdims=True)
        ii = lax.broadcasted_iota(jnp.int32, (nrows, NBINS), 1)
        t_best = jnp.min(jnp.where(icv == mxv, ii, NBINS),
                         axis=1, keepdims=True)
        any_valid = jnp.max(valid.astype(jnp.int32), axis=1,
                            keepdims=True) > 0
        mn = mm_ref[0:1, 0:1]
        width = mm_ref[1:2, 0:1]
        thr = mn + (t_best + 1).astype(jnp.float32) * width
        thr = jnp.where(any_valid, thr, jnp.float32(0.0))
        thr_ref[...] = jnp.broadcast_to(thr, (nrows, 128))

    return body


def _thresholds(hist_parts, mm, nrows, npix):
    return pl.pallas_call(
        _make_thresholds_body(nrows, npix),
        out_shape=jax.ShapeDtypeStruct((nrows, 128), jnp.float32),
    )(hist_parts, mm)


def _mask_body(x_ref, thr_ref, o_ref):
    xb = x_ref[...]
    thr = thr_ref[:, 0:1]
    o_ref[...] = jnp.where(xb <= thr, jnp.float32(0.0), xb)


def _mask(x_flat, thr, nrows, npix):
    ncols = 16384
    grid = npix // ncols
    return pl.pallas_call(
        _mask_body,
        grid=(grid,),
        in_specs=[
            pl.BlockSpec((nrows, ncols), lambda i: (0, i)),
            pl.BlockSpec((nrows, 128), lambda i: (0, 0)),
        ],
        out_specs=pl.BlockSpec((nrows, ncols), lambda i: (0, i)),
        out_shape=jax.ShapeDtypeStruct((nrows, npix), jnp.float32),
    )(x_flat, thr)


def kernel(x):
    b, c, h, w = x.shape
    nrows = b * c
    npix = h * w
    x_flat = x.reshape(nrows, npix)
    mm = _minmax(x_flat, nrows, npix)
    hist_parts = _sc_hist(x_flat, mm, nrows, npix)
    thr = _thresholds(hist_parts.reshape(NWORKERS, nrows, NBINS),
                      mm, nrows, npix)
    out = _mask(x_flat, thr, nrows, npix)
    return out.reshape(x.shape)


# speedup vs baseline: 1.2019x; 1.1661x over previous
import dataclasses
import functools

import jax
import jax.numpy as jnp
from jax import lax
from jax.experimental import pallas as pl
from jax.experimental.pallas import tpu as pltpu
from jax.experimental.pallas import tpu_sc as plsc

NBINS = 256
NWORKERS = 32


def _make_minmax_body(c):
    def body(x_ref, o_ref, amn_ref, amx_ref):
        i = pl.program_id(0)
        cmn = x_ref[0, 0]
        cmx = cmn
        for k in range(1, c):
            cmn = jnp.minimum(cmn, x_ref[0, k])
            cmx = jnp.maximum(cmx, x_ref[0, k])

        @pl.when(i == 0)
        def _init():
            amn_ref[...] = cmn
            amx_ref[...] = cmx

        @pl.when(i > 0)
        def _acc():
            amn_ref[...] = jnp.minimum(amn_ref[...], cmn)
            amx_ref[...] = jnp.maximum(amx_ref[...], cmx)

        @pl.when(i == pl.num_programs(0) - 1)
        def _emit():
            mn = jnp.min(amn_ref[...])
            mx = jnp.max(amx_ref[...])
            width = (mx - mn) / jnp.float32(NBINS)
            row = lax.broadcasted_iota(jnp.int32, (8, 128), 0)
            o_ref[...] = jnp.where(row == 0, mn, width)

    return body


def _minmax(x):
    b, c, h, w = x.shape
    return pl.pallas_call(
        _make_minmax_body(c),
        grid=(b,),
        in_specs=[pl.BlockSpec((1, c, h, w), lambda i: (i, 0, 0, 0))],
        out_specs=pl.BlockSpec((8, 128), lambda i: (0, 0)),
        out_shape=jax.ShapeDtypeStruct((8, 128), jnp.float32),
        scratch_shapes=[pltpu.VMEM((h, w), jnp.float32),
                        pltpu.VMEM((h, w), jnp.float32)],
    )(x)


def _sc_hist_body(nrows, chunk, x_hbm, mm_hbm, hist_hbm,
                  mm_v, buf0_v, buf1_v, hist_v, sem_a, sem_b):
    wid = lax.axis_index("s") * 2 + lax.axis_index("c")
    base = wid * chunk

    zeros16 = jnp.zeros((16,), jnp.float32)

    @pl.loop(0, nrows * NBINS, step=16)
    def _zero(i):
        hist_v[pl.ds(i, 16)] = zeros16

    pltpu.sync_copy(mm_hbm, mm_v)
    mn_v = mm_v[0, pl.ds(0, 16)]
    w_v = mm_v[1, pl.ds(0, 16)]
    rw_v = jnp.float32(1.0) / w_v

    ones16 = jnp.ones((16,), jnp.float32)
    top = jnp.full((16,), NBINS - 1, jnp.int32)
    bot = jnp.zeros((16,), jnp.int32)

    sems = (sem_a, sem_b)
    bufs = (buf0_v, buf1_v)
    cp = pltpu.async_copy(x_hbm.at[0, pl.ds(base, chunk)], bufs[0], sems[0])
    for r in range(nrows):
        cur = r % 2
        cp.wait()
        if r + 1 < nrows:
            cp = pltpu.async_copy(
                x_hbm.at[r + 1, pl.ds(base, chunk)],
                bufs[1 - cur], sems[1 - cur])
        rbase = jnp.full((16,), r * NBINS, jnp.int32)
        bufr = bufs[cur]

        @pl.loop(0, chunk, step=128)
        def _bin(i):
            for u in range(8):
                v = bufr[pl.ds(i + u * 16, 16)]
                q = (v - mn_v) / w_v
                b = q.astype(jnp.int32)
                b = jnp.minimum(jnp.maximum(b, bot), top) + rbase
                plsc.addupdate_scatter(hist_v, [b], ones16)

    pltpu.sync_copy(hist_v, hist_hbm.at[wid])


def _sc_hist(x_flat, mm, nrows, npix):
    chunk = npix // NWORKERS
    mesh = plsc.VectorSubcoreMesh(core_axis_name="c", subcore_axis_name="s")
    cp = pltpu.CompilerParams()
    if "needs_layout_passes" in pltpu.CompilerParams.__dataclass_fields__:
        cp = dataclasses.replace(cp, needs_layout_passes=False)
    kern = functools.partial(
        pl.kernel,
        compiler_params=cp,
        out_type=jax.ShapeDtypeStruct((NWORKERS, nrows * NBINS), jnp.float32),
        mesh=mesh,
        scratch_types=[
            pltpu.VMEM((8, 128), jnp.float32),
            pltpu.VMEM((chunk,), jnp.float32),
            pltpu.VMEM((chunk,), jnp.float32),
            pltpu.VMEM((nrows * NBINS,), jnp.float32),
            pltpu.SemaphoreType.DMA,
            pltpu.SemaphoreType.DMA,
        ],
    )(functools.partial(_sc_hist_body, nrows, chunk))
    return kern(x_flat, mm)


def _make_thresholds_body(nrows, npix):
    inv_n = 1.0 / float(npix)

    def body(hist_ref, mm_ref, thr_ref):
        h = hist_ref[0]
        for k in range(1, NWORKERS):
            h = h + hist_ref[k]
        hn = h * jnp.float32(inv_n)
        tv = lax.broadcasted_iota(jnp.int32, (1, NBINS), 1).astype(jnp.float32)
        bi = lax.broadcasted_iota(jnp.int32, (NBINS, NBINS), 0)
        ti = lax.broadcasted_iota(jnp.int32, (NBINS, NBINS), 1)
        cum = (bi <= ti).astype(jnp.float32)
        w_bg = jnp.dot(hn, cum, precision=lax.Precision.HIGHEST)
        s_bg = jnp.dot(hn * tv, cum, precision=lax.Precision.HIGHEST)
        total = s_bg[:, NBINS - 1:NBINS]
        w_fg = 1.0 - w_bg
        valid = (w_bg != 0.0) & (w_fg != 0.0)
        safe_w_bg = jnp.where(valid, w_bg, jnp.float32(1.0))
        safe_w_fg = jnp.where(valid, w_fg, jnp.float32(1.0))
        mean_bg = s_bg / safe_w_bg
        mean_fg = (total - s_bg) / safe_w_fg
        icv = w_bg * w_fg * (mean_bg - mean_fg) ** 2
        icv = jnp.where(valid, icv, -jnp.inf)
        mxv = jnp.max(icv, axis=1, keepdims=True)
        ii = lax.broadcasted_iota(jnp.int32, (nrows, NBINS), 1)
        t_best = jnp.min(jnp.where(icv == mxv, ii, NBINS),
                         axis=1, keepdims=True)
        any_valid = jnp.max(valid.astype(jnp.int32), axis=1,
                            keepdims=True) > 0
        mn = mm_ref[0:1, 0:1]
        width = mm_ref[1:2, 0:1]
        thr = mn + (t_best + 1).astype(jnp.float32) * width
        thr = jnp.where(any_valid, thr, jnp.float32(0.0))
        thr_ref[...] = jnp.broadcast_to(thr, (nrows, 128))

    return body


def _thresholds(hist_parts, mm, nrows, npix):
    return pl.pallas_call(
        _make_thresholds_body(nrows, npix),
        out_shape=jax.ShapeDtypeStruct((nrows, 128), jnp.float32),
    )(hist_parts, mm)


def _make_mask_body(c):
    def body(x_ref, thr_ref, o_ref):
        i = pl.program_id(0)
        for k in range(c):
            xk = x_ref[0, k]
            tk = thr_ref[pl.ds(i * c + k, 1), 0:1]
            o_ref[0, k] = jnp.where(xk <= tk, jnp.float32(0.0), xk)

    return body


def _mask(x, thr):
    b, c, h, w = x.shape
    return pl.pallas_call(
        _make_mask_body(c),
        grid=(b,),
        in_specs=[
            pl.BlockSpec((1, c, h, w), lambda i: (i, 0, 0, 0)),
            pl.BlockSpec((b * c, 128), lambda i: (0, 0)),
        ],
        out_specs=pl.BlockSpec((1, c, h, w), lambda i: (i, 0, 0, 0)),
        out_shape=jax.ShapeDtypeStruct(x.shape, jnp.float32),
    )(x, thr)


def kernel(x):
    b, c, h, w = x.shape
    nrows = b * c
    npix = h * w
    x_flat = x.reshape(nrows, npix)
    mm = _minmax(x)
    hist_parts = _sc_hist(x_flat, mm, nrows, npix)
    thr = _thresholds(hist_parts.reshape(NWORKERS, nrows, NBINS),
                      mm, nrows, npix)
    out = _mask(x, thr)
    return out


# speedup vs baseline: 2.5954x; 2.1595x over previous
import dataclasses
import functools

import jax
import jax.numpy as jnp
from jax import lax
from jax.experimental import pallas as pl
from jax.experimental.pallas import tpu as pltpu
from jax.experimental.pallas import tpu_sc as plsc

NBINS = 256
NWORKERS = 32


def _make_minmax_body(c):
    def body(x_ref, o_ref, amn_ref, amx_ref):
        i = pl.program_id(0)
        cmn = x_ref[0, 0]
        cmx = cmn
        for k in range(1, c):
            cmn = jnp.minimum(cmn, x_ref[0, k])
            cmx = jnp.maximum(cmx, x_ref[0, k])

        @pl.when(i == 0)
        def _init():
            amn_ref[...] = cmn
            amx_ref[...] = cmx

        @pl.when(i > 0)
        def _acc():
            amn_ref[...] = jnp.minimum(amn_ref[...], cmn)
            amx_ref[...] = jnp.maximum(amx_ref[...], cmx)

        @pl.when(i == pl.num_programs(0) - 1)
        def _emit():
            mn = jnp.min(amn_ref[...])
            mx = jnp.max(amx_ref[...])
            width = (mx - mn) / jnp.float32(NBINS)
            row = lax.broadcasted_iota(jnp.int32, (8, 128), 0)
            o_ref[...] = jnp.where(row == 0, mn, width)

    return body


def _minmax(x):
    b, c, h, w = x.shape
    return pl.pallas_call(
        _make_minmax_body(c),
        grid=(b,),
        in_specs=[pl.BlockSpec((1, c, h, w), lambda i: (i, 0, 0, 0))],
        out_specs=pl.BlockSpec((8, 128), lambda i: (0, 0)),
        out_shape=jax.ShapeDtypeStruct((8, 128), jnp.float32),
        scratch_shapes=[pltpu.VMEM((h, w), jnp.float32),
                        pltpu.VMEM((h, w), jnp.float32)],
    )(x)


def _sc_hist_body(nrows, chunk, x_hbm, mm_hbm, hist_hbm,
                  mm_v, buf0_v, buf1_v, hist_v, sem_a, sem_b):
    wid = lax.axis_index("s") * 2 + lax.axis_index("c")
    base = wid * chunk

    zeros16 = jnp.zeros((16,), jnp.float32)

    @pl.loop(0, nrows * NBINS, step=16)
    def _zero(i):
        hist_v[pl.ds(i, 16)] = zeros16

    pltpu.sync_copy(mm_hbm, mm_v)
    mn_v = mm_v[0, pl.ds(0, 16)]
    w_v = mm_v[1, pl.ds(0, 16)]
    rw_v = jnp.float32(1.0) / w_v

    ones16 = jnp.ones((16,), jnp.float32)
    top = jnp.full((16,), NBINS - 1, jnp.int32)
    bot = jnp.zeros((16,), jnp.int32)

    sems = (sem_a, sem_b)
    bufs = (buf0_v, buf1_v)
    cp = pltpu.async_copy(x_hbm.at[0, pl.ds(base, chunk)], bufs[0], sems[0])
    for r in range(nrows):
        cur = r % 2
        cp.wait()
        if r + 1 < nrows:
            cp = pltpu.async_copy(
                x_hbm.at[r + 1, pl.ds(base, chunk)],
                bufs[1 - cur], sems[1 - cur])
        rbase = jnp.full((16,), r * NBINS, jnp.int32)
        bufr = bufs[cur]

        @plsc.parallel_loop(0, chunk, step=128)
        def _bin(i):
            for u in range(8):
                v = bufr[pl.ds(i + u * 16, 16)]
                q = (v - mn_v) / w_v
                b = q.astype(jnp.int32)
                b = jnp.minimum(jnp.maximum(b, bot), top) + rbase
                plsc.addupdate_scatter(hist_v, [b], ones16)

    pltpu.sync_copy(hist_v, hist_hbm.at[wid])


def _sc_hist(x_flat, mm, nrows, npix):
    chunk = npix // NWORKERS
    mesh = plsc.VectorSubcoreMesh(core_axis_name="c", subcore_axis_name="s")
    cp = pltpu.CompilerParams()
    if "needs_layout_passes" in pltpu.CompilerParams.__dataclass_fields__:
        cp = dataclasses.replace(cp, needs_layout_passes=False)
    kern = functools.partial(
        pl.kernel,
        compiler_params=cp,
        out_type=jax.ShapeDtypeStruct((NWORKERS, nrows * NBINS), jnp.float32),
        mesh=mesh,
        scratch_types=[
            pltpu.VMEM((8, 128), jnp.float32),
            pltpu.VMEM((chunk,), jnp.float32),
            pltpu.VMEM((chunk,), jnp.float32),
            pltpu.VMEM((nrows * NBINS,), jnp.float32),
            pltpu.SemaphoreType.DMA,
            pltpu.SemaphoreType.DMA,
        ],
    )(functools.partial(_sc_hist_body, nrows, chunk))
    return kern(x_flat, mm)


def _make_thresholds_body(nrows, npix):
    inv_n = 1.0 / float(npix)

    def body(hist_ref, mm_ref, thr_ref):
        h = hist_ref[0]
        for k in range(1, NWORKERS):
            h = h + hist_ref[k]
        hn = h * jnp.float32(inv_n)
        tv = lax.broadcasted_iota(jnp.int32, (1, NBINS), 1).astype(jnp.float32)
        bi = lax.broadcasted_iota(jnp.int32, (NBINS, NBINS), 0)
        ti = lax.broadcasted_iota(jnp.int32, (NBINS, NBINS), 1)
        cum = (bi <= ti).astype(jnp.float32)
        w_bg = jnp.dot(hn, cum, precision=lax.Precision.HIGHEST)
        s_bg = jnp.dot(hn * tv, cum, precision=lax.Precision.HIGHEST)
        total = s_bg[:, NBINS - 1:NBINS]
        w_fg = 1.0 - w_bg
        valid = (w_bg != 0.0) & (w_fg != 0.0)
        safe_w_bg = jnp.where(valid, w_bg, jnp.float32(1.0))
        safe_w_fg = jnp.where(valid, w_fg, jnp.float32(1.0))
        mean_bg = s_bg / safe_w_bg
        mean_fg = (total - s_bg) / safe_w_fg
        icv = w_bg * w_fg * (mean_bg - mean_fg) ** 2
        icv = jnp.where(valid, icv, -jnp.inf)
        mxv = jnp.max(icv, axis=1, keepdims=True)
        ii = lax.broadcasted_iota(jnp.int32, (nrows, NBINS), 1)
        t_best = jnp.min(jnp.where(icv == mxv, ii, NBINS),
                         axis=1, keepdims=True)
        any_valid = jnp.max(valid.astype(jnp.int32), axis=1,
                            keepdims=True) > 0
        mn = mm_ref[0:1, 0:1]
        width = mm_ref[1:2, 0:1]
        thr = mn + (t_best + 1).astype(jnp.float32) * width
        thr = jnp.where(any_valid, thr, jnp.float32(0.0))
        thr_ref[...] = jnp.broadcast_to(thr, (nrows, 128))

    return body


def _thresholds(hist_parts, mm, nrows, npix):
    return pl.pallas_call(
        _make_thresholds_body(nrows, npix),
        out_shape=jax.ShapeDtypeStruct((nrows, 128), jnp.float32),
    )(hist_parts, mm)


def _make_mask_body(c):
    def body(x_ref, thr_ref, o_ref):
        i = pl.program_id(0)
        for k in range(c):
            xk = x_ref[0, k]
            tk = thr_ref[pl.ds(i * c + k, 1), 0:1]
            o_ref[0, k] = jnp.where(xk <= tk, jnp.float32(0.0), xk)

    return body


def _mask(x, thr):
    b, c, h, w = x.shape
    return pl.pallas_call(
        _make_mask_body(c),
        grid=(b,),
        in_specs=[
            pl.BlockSpec((1, c, h, w), lambda i: (i, 0, 0, 0)),
            pl.BlockSpec((b * c, 128), lambda i: (0, 0)),
        ],
        out_specs=pl.BlockSpec((1, c, h, w), lambda i: (i, 0, 0, 0)),
        out_shape=jax.ShapeDtypeStruct(x.shape, jnp.float32),
    )(x, thr)


def kernel(x):
    b, c, h, w = x.shape
    nrows = b * c
    npix = h * w
    x_flat = x.reshape(nrows, npix)
    mm = _minmax(x)
    hist_parts = _sc_hist(x_flat, mm, nrows, npix)
    thr = _thresholds(hist_parts.reshape(NWORKERS, nrows, NBINS),
                      mm, nrows, npix)
    out = _mask(x, thr)
    return out
